# baseline (speedup 1.0000x reference)
import functools

import jax
import jax.numpy as jnp
from jax import lax
from jax.experimental import pallas as pl
from jax.experimental.pallas import tpu as pltpu
from jax.experimental.pallas import tpu_sc as plsc

_info = plsc.get_sparse_core_info()
_NC, _NS, _L = _info.num_cores, _info.num_subcores, _info.num_lanes
_NW = _NC * _NS

_B = 1024
_LAB_PER_W = _B // _NW
_OUT_PER_W = 2 * _LAB_PER_W


def _sc_onehot(labels_hbm, out_hbm, lab_v, idx_v, ones_v, zero_v, sem):
    wid = lax.axis_index("s") * _NC + lax.axis_index("c")
    base = wid * _LAB_PER_W
    pltpu.sync_copy(labels_hbm.at[pl.ds(base, _LAB_PER_W)], lab_v)
    iota = lax.iota(jnp.int32, _L)
    zeros = jnp.zeros((_L,), jnp.float32)
    ones = jnp.ones((_L,), jnp.float32)
    for k in range(_LAB_PER_W // _L):
        lab = lab_v[pl.ds(k * _L, _L)]
        idx_v[pl.ds(k * _L, _L)] = 2 * (base + k * _L + iota) + lab
        ones_v[pl.ds(k * _L, _L)] = ones
    for k in range(_OUT_PER_W // _L):
        zero_v[pl.ds(k * _L, _L)] = zeros
    pltpu.sync_copy(zero_v, out_hbm.at[pl.ds(2 * base, _OUT_PER_W)])
    pltpu.async_copy(ones_v, out_hbm.at[idx_v], sem).wait()


@functools.partial(
    pl.kernel,
    mesh=plsc.VectorSubcoreMesh(core_axis_name="c", subcore_axis_name="s"),
    out_type=jax.ShapeDtypeStruct((2 * _B,), jnp.float32),
    scratch_types=[
        pltpu.VMEM((_LAB_PER_W,), jnp.int32),
        pltpu.VMEM((_LAB_PER_W,), jnp.int32),
        pltpu.VMEM((_LAB_PER_W,), jnp.float32),
        pltpu.VMEM((_OUT_PER_W,), jnp.float32),
        pltpu.SemaphoreType.DMA,
    ],
)
def _sc_call(labels_hbm, out_hbm, lab_v, idx_v, ones_v, zero_v, sem):
    _sc_onehot(labels_hbm, out_hbm, lab_v, idx_v, ones_v, zero_v, sem)


def kernel(input_ids, attention_mask, labels):
    batch = input_ids.shape[0]
    return _sc_call(labels).reshape(batch, 2)


# speedup vs baseline: 1.0067x; 1.0067x over previous
import functools

import jax
import jax.numpy as jnp
from jax import lax
from jax.experimental import pallas as pl
from jax.experimental.pallas import tpu as pltpu
from jax.experimental.pallas import tpu_sc as plsc

_info = plsc.get_sparse_core_info()
_NC, _NS, _L = _info.num_cores, _info.num_subcores, _info.num_lanes
_NW = _NC * _NS

_B = 1024
_LAB_PER_W = _B // _NW
_OUT_PER_W = 2 * _LAB_PER_W


def _sc_onehot(labels_hbm, out_hbm, lab_v, idx_v, ones_v, zero_v, lab_sem, zero_sem, scat_sem):
    wid = lax.axis_index("s") * _NC + lax.axis_index("c")
    base = wid * _LAB_PER_W
    iota = lax.iota(jnp.int32, _L)
    zeros = jnp.zeros((_L,), jnp.float32)
    ones = jnp.ones((_L,), jnp.float32)
    lab_cp = pltpu.async_copy(labels_hbm.at[pl.ds(base, _LAB_PER_W)], lab_v, lab_sem)
    for k in range(_OUT_PER_W // _L):
        zero_v[pl.ds(k * _L, _L)] = zeros
    zero_cp = pltpu.async_copy(zero_v, out_hbm.at[pl.ds(2 * base, _OUT_PER_W)], zero_sem)
    lab_cp.wait()
    for k in range(_LAB_PER_W // _L):
        lab = lab_v[pl.ds(k * _L, _L)]
        idx_v[pl.ds(k * _L, _L)] = 2 * (base + k * _L + iota) + lab
        ones_v[pl.ds(k * _L, _L)] = ones
    zero_cp.wait()
    pltpu.async_copy(ones_v, out_hbm.at[idx_v], scat_sem).wait()


@functools.partial(
    pl.kernel,
    mesh=plsc.VectorSubcoreMesh(core_axis_name="c", subcore_axis_name="s"),
    out_type=jax.ShapeDtypeStruct((2 * _B,), jnp.float32),
    scratch_types=[
        pltpu.VMEM((_LAB_PER_W,), jnp.int32),
        pltpu.VMEM((_LAB_PER_W,), jnp.int32),
        pltpu.VMEM((_LAB_PER_W,), jnp.float32),
        pltpu.VMEM((_OUT_PER_W,), jnp.float32),
        pltpu.SemaphoreType.DMA,
        pltpu.SemaphoreType.DMA,
        pltpu.SemaphoreType.DMA,
    ],
)
def _sc_call(labels_hbm, out_hbm, lab_v, idx_v, ones_v, zero_v, lab_sem, zero_sem, scat_sem):
    _sc_onehot(labels_hbm, out_hbm, lab_v, idx_v, ones_v, zero_v, lab_sem, zero_sem, scat_sem)


def kernel(input_ids, attention_mask, labels):
    batch = input_ids.shape[0]
    return _sc_call(labels).reshape(batch, 2)


# speedup vs baseline: 1.0509x; 1.0439x over previous
import functools

import jax
import jax.numpy as jnp
from jax import lax
from jax.experimental import pallas as pl
from jax.experimental.pallas import tpu as pltpu
from jax.experimental.pallas import tpu_sc as plsc

_info = plsc.get_sparse_core_info()
_NC, _NS, _L = 1, _info.num_subcores, _info.num_lanes
_NW = _NC * _NS

_B = 1024
_LAB_PER_W = _B // _NW
_OUT_PER_W = 2 * _LAB_PER_W


def _sc_onehot(labels_hbm, out_hbm, lab_v, idx_v, ones_v, zero_v, lab_sem, zero_sem, scat_sem):
    wid = lax.axis_index("s") * _NC + lax.axis_index("c")
    base = wid * _LAB_PER_W
    iota = lax.iota(jnp.int32, _L)
    zeros = jnp.zeros((_L,), jnp.float32)
    ones = jnp.ones((_L,), jnp.float32)
    lab_cp = pltpu.async_copy(labels_hbm.at[pl.ds(base, _LAB_PER_W)], lab_v, lab_sem)
    for k in range(_OUT_PER_W // _L):
        zero_v[pl.ds(k * _L, _L)] = zeros
    zero_cp = pltpu.async_copy(zero_v, out_hbm.at[pl.ds(2 * base, _OUT_PER_W)], zero_sem)
    lab_cp.wait()
    for k in range(_LAB_PER_W // _L):
        lab = lab_v[pl.ds(k * _L, _L)]
        idx_v[pl.ds(k * _L, _L)] = 2 * (base + k * _L + iota) + lab
        ones_v[pl.ds(k * _L, _L)] = ones
    zero_cp.wait()
    pltpu.async_copy(ones_v, out_hbm.at[idx_v], scat_sem).wait()


@functools.partial(
    pl.kernel,
    mesh=plsc.VectorSubcoreMesh(
        core_axis_name="c", subcore_axis_name="s", num_cores=1
    ),
    out_type=jax.ShapeDtypeStruct((2 * _B,), jnp.float32),
    scratch_types=[
        pltpu.VMEM((_LAB_PER_W,), jnp.int32),
        pltpu.VMEM((_LAB_PER_W,), jnp.int32),
        pltpu.VMEM((_LAB_PER_W,), jnp.float32),
        pltpu.VMEM((_OUT_PER_W,), jnp.float32),
        pltpu.SemaphoreType.DMA,
        pltpu.SemaphoreType.DMA,
        pltpu.SemaphoreType.DMA,
    ],
)
def _sc_call(labels_hbm, out_hbm, lab_v, idx_v, ones_v, zero_v, lab_sem, zero_sem, scat_sem):
    _sc_onehot(labels_hbm, out_hbm, lab_v, idx_v, ones_v, zero_v, lab_sem, zero_sem, scat_sem)


def kernel(input_ids, attention_mask, labels):
    batch = input_ids.shape[0]
    return _sc_call(labels).reshape(batch, 2)
